# baseline (speedup 1.0000x reference)
import functools

import jax
import jax.numpy as jnp
from jax import lax
from jax.experimental import pallas as pl
from jax.experimental.pallas import tpu as pltpu
from jax.experimental.pallas import tpu_sc as plsc

N_FEATURES = 32768
D_MODEL = 2048
BATCH = 16384
K = 32
U_ROWS = 8192
GRAD_CLIP_THRESHOLD = 1.0

NW = 32
ROWS_W = BATCH // NW
_UNROLL = 8

SC_UROWS = 4096
_SCU_W = SC_UROWS // NW
_CHUNK_R = 8
_NCHUNK = _SCU_W // _CHUNK_R
_CHUNK_E = _CHUNK_R * D_MODEL


def _make_hist_kernel():
    mesh = plsc.VectorSubcoreMesh(core_axis_name="c", subcore_axis_name="s")

    @functools.partial(
        pl.kernel,
        out_type=[
            jax.ShapeDtypeStruct((NW * N_FEATURES,), jnp.float32),
            jax.ShapeDtypeStruct((NW, 16), jnp.float32),
        ],
        mesh=mesh,
        scratch_types=[
            pltpu.VMEM((ROWS_W // 2, K), jnp.int32),
            pltpu.VMEM((N_FEATURES,), jnp.float32),
            pltpu.VMEM((_CHUNK_R, D_MODEL), jnp.float32),
            pltpu.VMEM((_CHUNK_R, D_MODEL), jnp.float32),
            pltpu.VMEM((16,), jnp.float32),
            pltpu.SemaphoreType.DMA,
            pltpu.SemaphoreType.DMA,
        ],
        compiler_params=pltpu.CompilerParams(
            needs_layout_passes=False, skip_device_barrier=True
        ),
    )
    def hist_kernel(
        idx_hbm, upd_hbm, hist_out, cnt_out, idx_v, hist_v, ub0, ub1, cnt_v, sem0, sem1
    ):
        wid = lax.axis_index("s") * 2 + lax.axis_index("c")
        zero = jnp.zeros((16,), jnp.float32)
        one = jnp.ones((16,), jnp.float32)
        thr = jnp.full((16,), GRAD_CLIP_THRESHOLD, jnp.float32)

        row0 = wid * _SCU_W
        bufs = (ub0, ub1)
        sems = (sem0, sem1)
        copies = [None, None]
        copies[0] = pltpu.async_copy(
            upd_hbm.at[pl.ds(row0, _CHUNK_R), :], ub0, sem0
        )

        def zbody(i, c):
            for j in range(_UNROLL):
                hist_v[pl.ds((i * _UNROLL + j) * 16, 16)] = zero
            return c

        lax.fori_loop(0, N_FEATURES // (16 * _UNROLL), zbody, 0)

        cnt = zero
        for ci in range(_NCHUNK):
            if ci + 1 < _NCHUNK:
                copies[(ci + 1) % 2] = pltpu.async_copy(
                    upd_hbm.at[pl.ds(row0 + (ci + 1) * _CHUNK_R, _CHUNK_R), :],
                    bufs[(ci + 1) % 2],
                    sems[(ci + 1) % 2],
                )
            copies[ci % 2].wait()
            buf = bufs[ci % 2]

            def cbody(r, acc0):
                def inner(k, acc):
                    for j in range(_UNROLL):
                        v = buf[r, pl.ds((k * _UNROLL + j) * 16, 16)]
                        acc = acc + jnp.where(jnp.abs(v) > thr, one, zero)
                    return acc

                return lax.fori_loop(0, D_MODEL // (16 * _UNROLL), inner, acc0)

            cnt = lax.fori_loop(0, _CHUNK_R, cbody, cnt)

        cnt_v[...] = cnt
        pltpu.sync_copy(cnt_v, cnt_out.at[wid])

        def body(i, c):
            for r in range(4):
                for j in range(K // 16):
                    vec = idx_v[i * 4 + r, pl.ds(j * 16, 16)]
                    plsc.addupdate_scatter(hist_v, [vec], one)
            return c

        for ph in range(2):
            pltpu.sync_copy(
                idx_hbm.at[pl.ds(wid * ROWS_W + ph * (ROWS_W // 2), ROWS_W // 2), :],
                idx_v,
            )
            lax.fori_loop(0, ROWS_W // 2 // 4, body, 0)

        pltpu.sync_copy(hist_v, hist_out.at[pl.ds(wid * N_FEATURES, N_FEATURES)])

    return hist_kernel


_X_BLOCK = 1024
_GRID = BATCH // _X_BLOCK
_U_BLOCK = (U_ROWS - SC_UROWS) // _GRID
_U_OFF = SC_UROWS // _U_BLOCK


def _reduce_body(w_ref, nw_ref, avg_ref, x_ref, u_ref, norm_ref, cnt_ref, acc_ref):
    i = pl.program_id(0)

    @pl.when(i == 0)
    def _init():
        acc_ref[0] = 0.0
        acc_ref[1] = 0.0

    xb = x_ref[...]
    rs = jnp.sum(xb * xb, axis=1, keepdims=True)
    nsum = jnp.sum(jnp.sqrt(rs))
    ub = u_ref[...]
    csum = jnp.sum((jnp.abs(ub) > GRAD_CLIP_THRESHOLD).astype(jnp.float32))
    acc_ref[0] += nsum
    acc_ref[1] += csum

    @pl.when(i == _GRID - 1)
    def _fini():
        norm_ref[0] = avg_ref[0] * w_ref[0] + (acc_ref[0] / BATCH) * nw_ref[0]
        cnt_ref[0] = acc_ref[1]


def _dense_reduce(w, nw, avg_norm, x, updates_flat):
    return pl.pallas_call(
        _reduce_body,
        grid=(_GRID,),
        in_specs=[
            pl.BlockSpec(memory_space=pltpu.SMEM),
            pl.BlockSpec(memory_space=pltpu.SMEM),
            pl.BlockSpec(memory_space=pltpu.SMEM),
            pl.BlockSpec((_X_BLOCK, D_MODEL), lambda i: (i, 0)),
            pl.BlockSpec((_U_BLOCK, D_MODEL), lambda i: (_U_OFF + i, 0)),
        ],
        out_specs=[
            pl.BlockSpec(memory_space=pltpu.SMEM),
            pl.BlockSpec(memory_space=pltpu.SMEM),
        ],
        out_shape=[
            jax.ShapeDtypeStruct((1,), jnp.float32),
            jax.ShapeDtypeStruct((1,), jnp.float32),
        ],
        scratch_shapes=[pltpu.SMEM((2,), jnp.float32)],
        compiler_params=pltpu.CompilerParams(
            dimension_semantics=("arbitrary",)
        ),
    )(w, nw, avg_norm, x, updates_flat)


_FD_ROWS = N_FEATURES // 128


def _blend_body(
    w_ref, nw_ref, gcp_ref, tc_cnt_ref, fd_ref, h_ref, sc_cnt_ref, out_ref, clip_ref
):
    h = h_ref[...].reshape(NW, _FD_ROWS, 128)
    tot = jnp.sum(h, axis=0)
    out_ref[...] = fd_ref[...] * w_ref[0] + tot * nw_ref[0]
    total_cnt = tc_cnt_ref[0] + jnp.sum(sc_cnt_ref[...])
    clip_ref[0] = gcp_ref[0] * w_ref[0] + (
        total_cnt / (float(U_ROWS) * D_MODEL)
    ) * nw_ref[0]


def _blend(w, nw, gcp, tc_cnt, fd2, hists2, sc_cnt):
    return pl.pallas_call(
        _blend_body,
        in_specs=[
            pl.BlockSpec(memory_space=pltpu.SMEM),
            pl.BlockSpec(memory_space=pltpu.SMEM),
            pl.BlockSpec(memory_space=pltpu.SMEM),
            pl.BlockSpec(memory_space=pltpu.SMEM),
            pl.BlockSpec(memory_space=pltpu.VMEM),
            pl.BlockSpec(memory_space=pltpu.VMEM),
            pl.BlockSpec(memory_space=pltpu.VMEM),
        ],
        out_specs=[
            pl.BlockSpec(memory_space=pltpu.VMEM),
            pl.BlockSpec(memory_space=pltpu.SMEM),
        ],
        out_shape=[
            jax.ShapeDtypeStruct((_FD_ROWS, 128), jnp.float32),
            jax.ShapeDtypeStruct((1,), jnp.float32),
        ],
    )(w, nw, gcp, tc_cnt, fd2, hists2, sc_cnt)


def kernel(n_steps, avg_norm, feature_density, grad_clip_percent, updates_flat, x, k_indices):
    ns = jnp.asarray(n_steps, jnp.float32)
    w = (ns / (ns + 1.0)).reshape(1)
    nw = (1.0 / (ns + 1.0)).reshape(1)

    hist_kernel = _make_hist_kernel()
    hists1d, sc_cnt = hist_kernel(k_indices, updates_flat)

    updated_avg_norm, tc_cnt = _dense_reduce(
        w,
        nw,
        jnp.asarray(avg_norm, jnp.float32).reshape(1),
        x,
        updates_flat,
    )

    fd2 = feature_density.reshape(_FD_ROWS, 128)
    hists2 = hists1d.reshape(NW * _FD_ROWS, 128)
    updated_fd, updated_clip = _blend(
        w,
        nw,
        jnp.asarray(grad_clip_percent, jnp.float32).reshape(1),
        tc_cnt,
        fd2,
        hists2,
        sc_cnt,
    )

    return (
        jnp.asarray(n_steps + 1),
        updated_avg_norm[0],
        updated_fd.reshape(N_FEATURES),
        updated_clip[0],
    )


# speedup vs baseline: 1.0925x; 1.0925x over previous
import functools

import jax
import jax.numpy as jnp
from jax import lax
from jax.experimental import pallas as pl
from jax.experimental.pallas import tpu as pltpu
from jax.experimental.pallas import tpu_sc as plsc

N_FEATURES = 32768
D_MODEL = 2048
BATCH = 16384
K = 32
GRAD_CLIP_THRESHOLD = 1.0

NW = 32
ROWS_W = BATCH // NW
_UNROLL = 8


def _make_hist_kernel():
    mesh = plsc.VectorSubcoreMesh(core_axis_name="c", subcore_axis_name="s")

    @functools.partial(
        pl.kernel,
        out_type=jax.ShapeDtypeStruct((NW * N_FEATURES,), jnp.float32),
        mesh=mesh,
        scratch_types=[
            pltpu.VMEM((K, ROWS_W), jnp.int32),
            pltpu.VMEM((N_FEATURES,), jnp.float32),
        ],
        compiler_params=pltpu.CompilerParams(
            needs_layout_passes=False, skip_device_barrier=True
        ),
    )
    def hist_kernel(idx_hbm, out_hbm, idx_v, hist_v):
        wid = lax.axis_index("s") * 2 + lax.axis_index("c")
        zero = jnp.zeros((16,), jnp.float32)

        def zbody(i, c):
            for j in range(_UNROLL):
                hist_v[pl.ds((i * _UNROLL + j) * 16, 16)] = zero
            return c

        lax.fori_loop(0, N_FEATURES // (16 * _UNROLL), zbody, 0)

        pltpu.sync_copy(idx_hbm.at[:, pl.ds(wid * ROWS_W, ROWS_W)], idx_v)

        ones = jnp.ones((16,), jnp.float32)

        def body(r, c):
            def inner(k, c2):
                for j in range(_UNROLL):
                    vec = idx_v[r, pl.ds((k * _UNROLL + j) * 16, 16)]
                    plsc.addupdate_scatter(hist_v, [vec], ones)
                return c2

            return lax.fori_loop(0, ROWS_W // (16 * _UNROLL), inner, c)

        lax.fori_loop(0, K, body, 0)

        pltpu.sync_copy(hist_v, out_hbm.at[pl.ds(wid * N_FEATURES, N_FEATURES)])

    return hist_kernel


_X_BLOCK = 1024
_U_BLOCK = 512
_GRID = BATCH // _X_BLOCK


def _reduce_body(
    w_ref, nw_ref, avg_ref, gcp_ref, x_ref, u_ref, norm_ref, clip_ref, acc_ref
):
    i = pl.program_id(0)

    @pl.when(i == 0)
    def _init():
        acc_ref[0] = 0.0
        acc_ref[1] = 0.0

    xb = x_ref[...]
    rs = jnp.sum(xb * xb, axis=1, keepdims=True)
    nsum = jnp.sum(jnp.sqrt(rs))
    ub = u_ref[...]
    csum = jnp.sum((jnp.abs(ub) > GRAD_CLIP_THRESHOLD).astype(jnp.float32))
    acc_ref[0] += nsum
    acc_ref[1] += csum

    @pl.when(i == _GRID - 1)
    def _fini():
        w = w_ref[0]
        nw = nw_ref[0]
        norm_ref[0] = avg_ref[0] * w + (acc_ref[0] / BATCH) * nw
        clip_ref[0] = gcp_ref[0] * w + (
            acc_ref[1] / (8192.0 * D_MODEL)
        ) * nw


def _dense_reduce(w, nw, avg_norm, gcp, x, updates_flat):
    return pl.pallas_call(
        _reduce_body,
        grid=(_GRID,),
        in_specs=[
            pl.BlockSpec(memory_space=pltpu.SMEM),
            pl.BlockSpec(memory_space=pltpu.SMEM),
            pl.BlockSpec(memory_space=pltpu.SMEM),
            pl.BlockSpec(memory_space=pltpu.SMEM),
            pl.BlockSpec((_X_BLOCK, D_MODEL), lambda i: (i, 0)),
            pl.BlockSpec((_U_BLOCK, D_MODEL), lambda i: (i, 0)),
        ],
        out_specs=[
            pl.BlockSpec(memory_space=pltpu.SMEM),
            pl.BlockSpec(memory_space=pltpu.SMEM),
        ],
        out_shape=[
            jax.ShapeDtypeStruct((1,), jnp.float32),
            jax.ShapeDtypeStruct((1,), jnp.float32),
        ],
        scratch_shapes=[pltpu.SMEM((2,), jnp.float32)],
        compiler_params=pltpu.CompilerParams(
            dimension_semantics=("arbitrary",)
        ),
    )(w, nw, avg_norm, gcp, x, updates_flat)


_FD_ROWS = N_FEATURES // 128


def _blend_body(w_ref, nw_ref, fd_ref, h_ref, out_ref):
    h = h_ref[...].reshape(NW, _FD_ROWS, 128)
    tot = jnp.sum(h, axis=0)
    out_ref[...] = fd_ref[...] * w_ref[0] + tot * nw_ref[0]


def _blend(w, nw, fd2, hists2):
    return pl.pallas_call(
        _blend_body,
        in_specs=[
            pl.BlockSpec(memory_space=pltpu.SMEM),
            pl.BlockSpec(memory_space=pltpu.SMEM),
            pl.BlockSpec(memory_space=pltpu.VMEM),
            pl.BlockSpec(memory_space=pltpu.VMEM),
        ],
        out_shape=jax.ShapeDtypeStruct((_FD_ROWS, 128), jnp.float32),
    )(w, nw, fd2, hists2)


def kernel(n_steps, avg_norm, feature_density, grad_clip_percent, updates_flat, x, k_indices):
    ns = jnp.asarray(n_steps, jnp.float32)
    w = (ns / (ns + 1.0)).reshape(1)
    nw = (1.0 / (ns + 1.0)).reshape(1)

    hist_kernel = _make_hist_kernel()
    hists1d = hist_kernel(k_indices.T)

    updated_avg_norm, updated_clip = _dense_reduce(
        w,
        nw,
        jnp.asarray(avg_norm, jnp.float32).reshape(1),
        jnp.asarray(grad_clip_percent, jnp.float32).reshape(1),
        x,
        updates_flat,
    )

    fd2 = feature_density.reshape(_FD_ROWS, 128)
    hists2 = hists1d.reshape(NW * _FD_ROWS, 128)
    updated_fd = _blend(w, nw, fd2, hists2).reshape(N_FEATURES)

    return (
        jnp.asarray(n_steps + 1),
        updated_avg_norm[0],
        updated_fd,
        updated_clip[0],
    )


# speedup vs baseline: 1.0946x; 1.0019x over previous
import functools

import jax
import jax.numpy as jnp
from jax import lax
from jax.experimental import pallas as pl
from jax.experimental.pallas import tpu as pltpu
from jax.experimental.pallas import tpu_sc as plsc

N_FEATURES = 32768
D_MODEL = 2048
BATCH = 16384
K = 32
GRAD_CLIP_THRESHOLD = 1.0

NW = 32
ROWS_W = BATCH // NW
_UNROLL = 8


def _make_hist_kernel():
    mesh = plsc.VectorSubcoreMesh(core_axis_name="c", subcore_axis_name="s")

    @functools.partial(
        pl.kernel,
        out_type=jax.ShapeDtypeStruct((NW * N_FEATURES,), jnp.float32),
        mesh=mesh,
        scratch_types=[
            pltpu.VMEM((K, ROWS_W), jnp.int32),
            pltpu.VMEM((N_FEATURES,), jnp.float32),
        ],
        compiler_params=pltpu.CompilerParams(
            needs_layout_passes=False
        ),
    )
    def hist_kernel(idx_hbm, out_hbm, idx_v, hist_v):
        wid = lax.axis_index("s") * 2 + lax.axis_index("c")
        zero = jnp.zeros((16,), jnp.float32)

        def zbody(i, c):
            for j in range(_UNROLL):
                hist_v[pl.ds((i * _UNROLL + j) * 16, 16)] = zero
            return c

        lax.fori_loop(0, N_FEATURES // (16 * _UNROLL), zbody, 0)

        pltpu.sync_copy(idx_hbm.at[:, pl.ds(wid * ROWS_W, ROWS_W)], idx_v)

        ones = jnp.ones((16,), jnp.float32)

        def body(r, c):
            def inner(k, c2):
                for j in range(_UNROLL):
                    vec = idx_v[r, pl.ds((k * _UNROLL + j) * 16, 16)]
                    plsc.addupdate_scatter(hist_v, [vec], ones)
                return c2

            return lax.fori_loop(0, ROWS_W // (16 * _UNROLL), inner, c)

        lax.fori_loop(0, K, body, 0)

        pltpu.sync_copy(hist_v, out_hbm.at[pl.ds(wid * N_FEATURES, N_FEATURES)])

    return hist_kernel


_X_BLOCK = 1024
_U_BLOCK = 512
_GRID = BATCH // _X_BLOCK


def _reduce_body(
    w_ref, nw_ref, avg_ref, gcp_ref, x_ref, u_ref, norm_ref, clip_ref, acc_ref
):
    i = pl.program_id(0)

    @pl.when(i == 0)
    def _init():
        acc_ref[0] = 0.0
        acc_ref[1] = 0.0

    xb = x_ref[...]
    rs = jnp.sum(xb * xb, axis=1, keepdims=True)
    nsum = jnp.sum(jnp.sqrt(rs))
    ub = u_ref[...]
    csum = jnp.sum((jnp.abs(ub) > GRAD_CLIP_THRESHOLD).astype(jnp.float32))
    acc_ref[0] += nsum
    acc_ref[1] += csum

    @pl.when(i == _GRID - 1)
    def _fini():
        w = w_ref[0]
        nw = nw_ref[0]
        norm_ref[0] = avg_ref[0] * w + (acc_ref[0] / BATCH) * nw
        clip_ref[0] = gcp_ref[0] * w + (
            acc_ref[1] / (8192.0 * D_MODEL)
        ) * nw


def _dense_reduce(w, nw, avg_norm, gcp, x, updates_flat):
    return pl.pallas_call(
        _reduce_body,
        grid=(_GRID,),
        in_specs=[
            pl.BlockSpec(memory_space=pltpu.SMEM),
            pl.BlockSpec(memory_space=pltpu.SMEM),
            pl.BlockSpec(memory_space=pltpu.SMEM),
            pl.BlockSpec(memory_space=pltpu.SMEM),
            pl.BlockSpec((_X_BLOCK, D_MODEL), lambda i: (i, 0)),
            pl.BlockSpec((_U_BLOCK, D_MODEL), lambda i: (i, 0)),
        ],
        out_specs=[
            pl.BlockSpec(memory_space=pltpu.SMEM),
            pl.BlockSpec(memory_space=pltpu.SMEM),
        ],
        out_shape=[
            jax.ShapeDtypeStruct((1,), jnp.float32),
            jax.ShapeDtypeStruct((1,), jnp.float32),
        ],
        scratch_shapes=[pltpu.SMEM((2,), jnp.float32)],
        compiler_params=pltpu.CompilerParams(
            dimension_semantics=("arbitrary",)
        ),
    )(w, nw, avg_norm, gcp, x, updates_flat)


_FD_ROWS = N_FEATURES // 128


def _blend_body(w_ref, nw_ref, fd_ref, h_ref, out_ref):
    h = h_ref[...].reshape(NW, _FD_ROWS, 128)
    tot = jnp.sum(h, axis=0)
    out_ref[...] = fd_ref[...] * w_ref[0] + tot * nw_ref[0]


def _blend(w, nw, fd2, hists2):
    return pl.pallas_call(
        _blend_body,
        in_specs=[
            pl.BlockSpec(memory_space=pltpu.SMEM),
            pl.BlockSpec(memory_space=pltpu.SMEM),
            pl.BlockSpec(memory_space=pltpu.VMEM),
            pl.BlockSpec(memory_space=pltpu.VMEM),
        ],
        out_shape=jax.ShapeDtypeStruct((_FD_ROWS, 128), jnp.float32),
    )(w, nw, fd2, hists2)


def kernel(n_steps, avg_norm, feature_density, grad_clip_percent, updates_flat, x, k_indices):
    ns = jnp.asarray(n_steps, jnp.float32)
    w = (ns / (ns + 1.0)).reshape(1)
    nw = (1.0 / (ns + 1.0)).reshape(1)

    hist_kernel = _make_hist_kernel()
    hists1d = hist_kernel(k_indices.T)

    updated_avg_norm, updated_clip = _dense_reduce(
        w,
        nw,
        jnp.asarray(avg_norm, jnp.float32).reshape(1),
        jnp.asarray(grad_clip_percent, jnp.float32).reshape(1),
        x,
        updates_flat,
    )

    fd2 = feature_density.reshape(_FD_ROWS, 128)
    hists2 = hists1d.reshape(NW * _FD_ROWS, 128)
    updated_fd = _blend(w, nw, fd2, hists2).reshape(N_FEATURES)

    return (
        jnp.asarray(n_steps + 1),
        updated_avg_norm[0],
        updated_fd,
        updated_clip[0],
    )
